# baseline (speedup 1.0000x reference)
import functools

import jax
import jax.numpy as jnp
import numpy as np
from jax import lax
from jax.experimental import pallas as pl
from jax.experimental.pallas import tpu as pltpu
from jax.experimental.pallas import tpu_sc as plsc

_MARGIN = 6.0
_EPSILON = 2.0
_DIM = 128
_EMB_RANGE = (_MARGIN + _EPSILON) / _DIM
_BATCH = 16384
_ENT_D = 2 * _DIM

_NC = 2
_NS = 16
_NW = _NC * _NS

_CHUNK = 128
_NSPLIT = 2
_CB = _BATCH // _NSPLIT


def _pipelined_gather(table_hbm, idx_v, idx_off, out_hbm, base, n_rows, bufs,
                      gsem, wsem):
    n = n_rows // _CHUNK
    gh = [None] * n
    wh = [None] * n
    for c in range(n):
        if c >= 1:
            gh[c - 1].wait()
            wh[c - 1] = pltpu.async_copy(
                bufs[(c - 1) % 2],
                out_hbm.at[pl.ds(base + (c - 1) * _CHUNK, _CHUNK)], wsem)
        if c >= 2:
            wh[c - 2].wait()
        gh[c] = pltpu.async_copy(
            table_hbm.at[idx_v.at[pl.ds(idx_off + c * _CHUNK, _CHUNK)]],
            bufs[c % 2], gsem)
    gh[n - 1].wait()
    wh[n - 1] = pltpu.async_copy(
        bufs[(n - 1) % 2],
        out_hbm.at[pl.ds(base + (n - 1) * _CHUNK, _CHUNK)], wsem)
    if n >= 2:
        wh[n - 2].wait()
    wh[n - 1].wait()


def _sc_gather_kernel(ent_hbm, rel_hbm, ih_hbm, it_hbm, ir_hbm, out_ht,
                      out_rel, iht_v, ir_v, bh0, bh1, br0, br1, gsem, wsem):
    wid = lax.axis_index("s") * _NC + lax.axis_index("c")
    n_h = _CB // _NW
    n_r = _CB // _NW
    base_h = wid * n_h
    base_r = wid * n_r
    pltpu.sync_copy(ih_hbm.at[pl.ds(base_h, n_h)], iht_v.at[pl.ds(0, n_h)])
    pltpu.sync_copy(it_hbm.at[pl.ds(base_h, n_h)], iht_v.at[pl.ds(n_h, n_h)])
    pltpu.sync_copy(ir_hbm.at[pl.ds(base_r, n_r)], ir_v)
    _pipelined_gather(ent_hbm, iht_v, 0, out_ht, base_h, n_h, (bh0, bh1),
                      gsem, wsem)
    _pipelined_gather(ent_hbm, iht_v, n_h, out_ht, _CB + base_h, n_h,
                      (bh0, bh1), gsem, wsem)
    _pipelined_gather(rel_hbm, ir_v, 0, out_rel, base_r, n_r, (br0, br1),
                      gsem, wsem)


def _sc_gather(entity_embedding, relation_embedding, heads, tails, relations):
    mesh = plsc.VectorSubcoreMesh(core_axis_name="c", subcore_axis_name="s")
    run = pl.kernel(
        _sc_gather_kernel,
        out_type=(
            jax.ShapeDtypeStruct((2 * _CB, _ENT_D), jnp.float32),
            jax.ShapeDtypeStruct((_CB, _DIM), jnp.float32),
        ),
        mesh=mesh,
        scratch_types=[
            pltpu.VMEM(((2 * _CB) // _NW,), jnp.int32),
            pltpu.VMEM((_CB // _NW,), jnp.int32),
            pltpu.VMEM((_CHUNK, _ENT_D), jnp.float32),
            pltpu.VMEM((_CHUNK, _ENT_D), jnp.float32),
            pltpu.VMEM((_CHUNK, _DIM), jnp.float32),
            pltpu.VMEM((_CHUNK, _DIM), jnp.float32),
            pltpu.SemaphoreType.DMA,
            pltpu.SemaphoreType.DMA,
        ],
    )
    return run(entity_embedding, relation_embedding, heads, tails, relations)


_BB = 2048

_SIN_C = (9.99999600e-01, -1.66665526e-01, 8.33240285e-03, -1.98086298e-04,
          2.69971060e-06, -2.03620814e-08)
_COS_C = (9.99999989e-01, -4.99999891e-01, 4.16664892e-02, -1.38878034e-03,
          2.47698803e-05, -2.70789985e-07, 1.72449738e-09)


def _poly_sin(x, t):
    acc = jnp.float32(_SIN_C[-1])
    for c in _SIN_C[-2::-1]:
        acc = acc * t + jnp.float32(c)
    return x * acc


def _poly_cos(t):
    acc = jnp.float32(_COS_C[-1])
    for c in _COS_C[-2::-1]:
        acc = acc * t + jnp.float32(c)
    return acc


def _tc_score_kernel(h_ref, t_ref, r_ref, o_ref):
    re_h = h_ref[:, :_DIM]
    im_h = h_ref[:, _DIM:]
    re_t = t_ref[:, :_DIM]
    im_t = t_ref[:, _DIM:]
    phase = r_ref[...] * np.float32(np.pi / _EMB_RANGE)
    t2 = phase * phase
    re_r = _poly_cos(t2)
    im_r = _poly_sin(phase, t2)
    re_rot = re_h * re_r - im_h * im_r
    im_rot = re_h * im_r + im_h * re_r
    d_re = re_rot - re_t
    d_im = im_rot - im_t
    score = d_re * d_re + d_im * d_im
    acc = jnp.sum(score * score, axis=1)
    o_ref[...] = _MARGIN - jnp.sqrt(acc)


def _tc_score(ht, relg):
    nblk = _CB // _BB
    return pl.pallas_call(
        _tc_score_kernel,
        grid=(nblk,),
        in_specs=[
            pl.BlockSpec((_BB, _ENT_D), lambda i: (i, 0)),
            pl.BlockSpec((_BB, _ENT_D), lambda i: (i + nblk, 0)),
            pl.BlockSpec((_BB, _DIM), lambda i: (i, 0)),
        ],
        out_specs=pl.BlockSpec((_BB,), lambda i: (i,)),
        out_shape=jax.ShapeDtypeStruct((_CB,), jnp.float32),
        compiler_params=pltpu.CompilerParams(
            dimension_semantics=("parallel",)),
    )(ht, ht, relg)


@jax.jit
def kernel(heads, relations, tails, entity_embedding, relation_embedding):
    heads = heads.astype(jnp.int32)
    tails = tails.astype(jnp.int32)
    relations = relations.astype(jnp.int32)
    outs = []
    for k in range(_NSPLIT):
        sl = slice(k * _CB, (k + 1) * _CB)
        ht, relg = _sc_gather(entity_embedding, relation_embedding,
                              heads[sl], tails[sl], relations[sl])
        outs.append(_tc_score(ht, relg))
    return jnp.concatenate(outs)


# speedup vs baseline: 1.0767x; 1.0767x over previous
import functools

import jax
import jax.numpy as jnp
import numpy as np
from jax import lax
from jax.experimental import pallas as pl
from jax.experimental.pallas import tpu as pltpu
from jax.experimental.pallas import tpu_sc as plsc

_MARGIN = 6.0
_EPSILON = 2.0
_DIM = 128
_EMB_RANGE = (_MARGIN + _EPSILON) / _DIM
_BATCH = 16384
_ENT_D = 2 * _DIM

_NC = 2
_NS = 16
_NW = _NC * _NS

_CHUNK = 128
_ECHUNK = 64
_NSPLIT = 1
_CB = _BATCH // _NSPLIT


def _stream_gather(units, chunk, bufs, gsem, wsem, n_inflight=2):
    n = len(units)
    depth = len(bufs)

    def start_gather(u):
        table_hbm, idx_v, idx_off, _, _ = units[u]
        return pltpu.async_copy(
            table_hbm.at[idx_v.at[pl.ds(idx_off, chunk)]],
            bufs[u % depth], gsem)

    def start_write(u):
        _, _, _, out_hbm, out_off = units[u]
        return pltpu.async_copy(
            bufs[u % depth], out_hbm.at[pl.ds(out_off, chunk)], wsem)

    gh = [None] * n
    wh = [None] * n
    waited = set()
    for u in range(min(n_inflight, n)):
        gh[u] = start_gather(u)
    for u in range(n):
        gh[u].wait()
        wh[u] = start_write(u)
        v = u + n_inflight
        if v < n:
            if v - depth >= 0 and (v - depth) not in waited:
                wh[v - depth].wait()
                waited.add(v - depth)
            gh[v] = start_gather(v)
    for u in range(n):
        if u not in waited:
            wh[u].wait()


def _sc_gather_kernel(ent_hbm, rel_hbm, ih_hbm, it_hbm, ir_hbm, out_ht,
                      out_rel, iht_v, ir_v, eb, rb, gsem, wsem, rgsem, rwsem):
    wid = lax.axis_index("s") * _NC + lax.axis_index("c")
    n_h = _CB // _NW
    n_r = _CB // _NW
    base_h = wid * n_h
    base_r = wid * n_r
    pltpu.sync_copy(ih_hbm.at[pl.ds(base_h, n_h)], iht_v.at[pl.ds(0, n_h)])
    pltpu.sync_copy(it_hbm.at[pl.ds(base_h, n_h)], iht_v.at[pl.ds(n_h, n_h)])
    pltpu.sync_copy(ir_hbm.at[pl.ds(base_r, n_r)], ir_v)
    ent_units = [
        (ent_hbm, iht_v, c * _ECHUNK, out_ht,
         base_h + c * _ECHUNK if c < n_h // _ECHUNK
         else _CB + base_h + c * _ECHUNK - n_h)
        for c in range((2 * n_h) // _ECHUNK)
    ]
    rel_units = [
        (rel_hbm, ir_v, c * _CHUNK, out_rel, base_r + c * _CHUNK)
        for c in range(n_r // _CHUNK)
    ]
    _stream_gather(ent_units, _ECHUNK, eb, gsem, wsem)
    _stream_gather(rel_units, _CHUNK, rb, rgsem, rwsem)


def _sc_gather(entity_embedding, relation_embedding, heads, tails, relations):
    mesh = plsc.VectorSubcoreMesh(core_axis_name="c", subcore_axis_name="s")
    run = pl.kernel(
        _sc_gather_kernel,
        out_type=(
            jax.ShapeDtypeStruct((2 * _CB, _ENT_D), jnp.float32),
            jax.ShapeDtypeStruct((_CB, _DIM), jnp.float32),
        ),
        mesh=mesh,
        scratch_types=[
            pltpu.VMEM(((2 * _CB) // _NW,), jnp.int32),
            pltpu.VMEM((_CB // _NW,), jnp.int32),
            tuple(pltpu.VMEM((_ECHUNK, _ENT_D), jnp.float32)
                  for _ in range(4)),
            tuple(pltpu.VMEM((_CHUNK, _DIM), jnp.float32)
                  for _ in range(2)),
            pltpu.SemaphoreType.DMA,
            pltpu.SemaphoreType.DMA,
            pltpu.SemaphoreType.DMA,
            pltpu.SemaphoreType.DMA,
        ],
    )
    return run(entity_embedding, relation_embedding, heads, tails, relations)


_BB = 2048

_SIN_C = (9.99999600e-01, -1.66665526e-01, 8.33240285e-03, -1.98086298e-04,
          2.69971060e-06, -2.03620814e-08)
_COS_C = (9.99999989e-01, -4.99999891e-01, 4.16664892e-02, -1.38878034e-03,
          2.47698803e-05, -2.70789985e-07, 1.72449738e-09)


def _poly_sin(x, t):
    acc = jnp.float32(_SIN_C[-1])
    for c in _SIN_C[-2::-1]:
        acc = acc * t + jnp.float32(c)
    return x * acc


def _poly_cos(t):
    acc = jnp.float32(_COS_C[-1])
    for c in _COS_C[-2::-1]:
        acc = acc * t + jnp.float32(c)
    return acc


def _tc_score_kernel(h_ref, t_ref, r_ref, o_ref):
    re_h = h_ref[:, :_DIM]
    im_h = h_ref[:, _DIM:]
    re_t = t_ref[:, :_DIM]
    im_t = t_ref[:, _DIM:]
    phase = r_ref[...] * np.float32(np.pi / _EMB_RANGE)
    t2 = phase * phase
    re_r = _poly_cos(t2)
    im_r = _poly_sin(phase, t2)
    re_rot = re_h * re_r - im_h * im_r
    im_rot = re_h * im_r + im_h * re_r
    d_re = re_rot - re_t
    d_im = im_rot - im_t
    score = d_re * d_re + d_im * d_im
    acc = jnp.sum(score * score, axis=1)
    o_ref[...] = _MARGIN - jnp.sqrt(acc)


def _tc_score(ht, relg):
    nblk = _CB // _BB
    return pl.pallas_call(
        _tc_score_kernel,
        grid=(nblk,),
        in_specs=[
            pl.BlockSpec((_BB, _ENT_D), lambda i: (i, 0)),
            pl.BlockSpec((_BB, _ENT_D), lambda i: (i + nblk, 0)),
            pl.BlockSpec((_BB, _DIM), lambda i: (i, 0)),
        ],
        out_specs=pl.BlockSpec((_BB,), lambda i: (i,)),
        out_shape=jax.ShapeDtypeStruct((_CB,), jnp.float32),
        compiler_params=pltpu.CompilerParams(
            dimension_semantics=("parallel",)),
    )(ht, ht, relg)


@jax.jit
def kernel(heads, relations, tails, entity_embedding, relation_embedding):
    heads = heads.astype(jnp.int32)
    tails = tails.astype(jnp.int32)
    relations = relations.astype(jnp.int32)
    outs = []
    for k in range(_NSPLIT):
        sl = slice(k * _CB, (k + 1) * _CB)
        ht, relg = _sc_gather(entity_embedding, relation_embedding,
                              heads[sl], tails[sl], relations[sl])
        outs.append(_tc_score(ht, relg))
    return jnp.concatenate(outs)


# speedup vs baseline: 1.0778x; 1.0010x over previous
import functools

import jax
import jax.numpy as jnp
import numpy as np
from jax import lax
from jax.experimental import pallas as pl
from jax.experimental.pallas import tpu as pltpu
from jax.experimental.pallas import tpu_sc as plsc

_MARGIN = 6.0
_EPSILON = 2.0
_DIM = 128
_EMB_RANGE = (_MARGIN + _EPSILON) / _DIM
_BATCH = 16384
_ENT_D = 2 * _DIM

_NC = 2
_NS = 16
_NW = _NC * _NS

_CHUNK = 128
_ECHUNK = 64
_NSPLIT = 1
_CB = _BATCH // _NSPLIT


def _stream_gather(units, chunk, bufs, gsem, wsem, n_inflight=3):
    n = len(units)
    depth = len(bufs)
    n_inflight = min(n_inflight, depth)

    def start_gather(u):
        table_hbm, idx_v, idx_off, _, _ = units[u]
        return pltpu.async_copy(
            table_hbm.at[idx_v.at[pl.ds(idx_off, chunk)]],
            bufs[u % depth], gsem)

    def start_write(u):
        _, _, _, out_hbm, out_off = units[u]
        return pltpu.async_copy(
            bufs[u % depth], out_hbm.at[pl.ds(out_off, chunk)], wsem)

    gh = [None] * n
    wh = [None] * n
    waited = set()
    for u in range(min(n_inflight, n)):
        gh[u] = start_gather(u)
    for u in range(n):
        gh[u].wait()
        wh[u] = start_write(u)
        v = u + n_inflight
        if v < n:
            if v - depth >= 0 and (v - depth) not in waited:
                wh[v - depth].wait()
                waited.add(v - depth)
            gh[v] = start_gather(v)
    for u in range(n):
        if u not in waited:
            wh[u].wait()


def _sc_gather_kernel(ent_hbm, rel_hbm, ih_hbm, it_hbm, ir_hbm, out_ht,
                      out_rel, iht_v, ir_v, eb, rb, gsem, wsem, rgsem, rwsem):
    wid = lax.axis_index("s") * _NC + lax.axis_index("c")
    n_h = _CB // _NW
    n_r = _CB // _NW
    base_h = wid * n_h
    base_r = wid * n_r
    pltpu.sync_copy(ih_hbm.at[pl.ds(base_h, n_h)], iht_v.at[pl.ds(0, n_h)])
    pltpu.sync_copy(it_hbm.at[pl.ds(base_h, n_h)], iht_v.at[pl.ds(n_h, n_h)])
    pltpu.sync_copy(ir_hbm.at[pl.ds(base_r, n_r)], ir_v)
    ent_units = [
        (ent_hbm, iht_v, c * _ECHUNK, out_ht,
         base_h + c * _ECHUNK if c < n_h // _ECHUNK
         else _CB + base_h + c * _ECHUNK - n_h)
        for c in range((2 * n_h) // _ECHUNK)
    ]
    rel_units = [
        (rel_hbm, ir_v, c * _CHUNK, out_rel, base_r + c * _CHUNK)
        for c in range(n_r // _CHUNK)
    ]
    _stream_gather(ent_units, _ECHUNK, eb, gsem, wsem)
    _stream_gather(rel_units, _CHUNK, rb, rgsem, rwsem)


def _sc_gather(entity_embedding, relation_embedding, heads, tails, relations):
    mesh = plsc.VectorSubcoreMesh(core_axis_name="c", subcore_axis_name="s")
    run = pl.kernel(
        _sc_gather_kernel,
        out_type=(
            jax.ShapeDtypeStruct((2 * _CB, _ENT_D), jnp.float32),
            jax.ShapeDtypeStruct((_CB, _DIM), jnp.float32),
        ),
        mesh=mesh,
        scratch_types=[
            pltpu.VMEM(((2 * _CB) // _NW,), jnp.int32),
            pltpu.VMEM((_CB // _NW,), jnp.int32),
            tuple(pltpu.VMEM((_ECHUNK, _ENT_D), jnp.float32)
                  for _ in range(4)),
            tuple(pltpu.VMEM((_CHUNK, _DIM), jnp.float32)
                  for _ in range(2)),
            pltpu.SemaphoreType.DMA,
            pltpu.SemaphoreType.DMA,
            pltpu.SemaphoreType.DMA,
            pltpu.SemaphoreType.DMA,
        ],
    )
    return run(entity_embedding, relation_embedding, heads, tails, relations)


_BB = 2048

_SIN_C = (9.99999600e-01, -1.66665526e-01, 8.33240285e-03, -1.98086298e-04,
          2.69971060e-06, -2.03620814e-08)
_COS_C = (9.99999989e-01, -4.99999891e-01, 4.16664892e-02, -1.38878034e-03,
          2.47698803e-05, -2.70789985e-07, 1.72449738e-09)


def _poly_sin(x, t):
    acc = jnp.float32(_SIN_C[-1])
    for c in _SIN_C[-2::-1]:
        acc = acc * t + jnp.float32(c)
    return x * acc


def _poly_cos(t):
    acc = jnp.float32(_COS_C[-1])
    for c in _COS_C[-2::-1]:
        acc = acc * t + jnp.float32(c)
    return acc


def _tc_score_kernel(h_ref, t_ref, r_ref, o_ref):
    re_h = h_ref[:, :_DIM]
    im_h = h_ref[:, _DIM:]
    re_t = t_ref[:, :_DIM]
    im_t = t_ref[:, _DIM:]
    phase = r_ref[...] * np.float32(np.pi / _EMB_RANGE)
    t2 = phase * phase
    re_r = _poly_cos(t2)
    im_r = _poly_sin(phase, t2)
    re_rot = re_h * re_r - im_h * im_r
    im_rot = re_h * im_r + im_h * re_r
    d_re = re_rot - re_t
    d_im = im_rot - im_t
    score = d_re * d_re + d_im * d_im
    acc = jnp.sum(score * score, axis=1)
    o_ref[...] = _MARGIN - jnp.sqrt(acc)


def _tc_score(ht, relg):
    nblk = _CB // _BB
    return pl.pallas_call(
        _tc_score_kernel,
        grid=(nblk,),
        in_specs=[
            pl.BlockSpec((_BB, _ENT_D), lambda i: (i, 0)),
            pl.BlockSpec((_BB, _ENT_D), lambda i: (i + nblk, 0)),
            pl.BlockSpec((_BB, _DIM), lambda i: (i, 0)),
        ],
        out_specs=pl.BlockSpec((_BB,), lambda i: (i,)),
        out_shape=jax.ShapeDtypeStruct((_CB,), jnp.float32),
        compiler_params=pltpu.CompilerParams(
            dimension_semantics=("parallel",)),
    )(ht, ht, relg)


@jax.jit
def kernel(heads, relations, tails, entity_embedding, relation_embedding):
    heads = heads.astype(jnp.int32)
    tails = tails.astype(jnp.int32)
    relations = relations.astype(jnp.int32)
    outs = []
    for k in range(_NSPLIT):
        sl = slice(k * _CB, (k + 1) * _CB)
        ht, relg = _sc_gather(entity_embedding, relation_embedding,
                              heads[sl], tails[sl], relations[sl])
        outs.append(_tc_score(ht, relg))
    return jnp.concatenate(outs)


# speedup vs baseline: 1.3000x; 1.2062x over previous
import functools

import jax
import jax.numpy as jnp
import numpy as np
from jax import lax
from jax.experimental import pallas as pl
from jax.experimental.pallas import tpu as pltpu
from jax.experimental.pallas import tpu_sc as plsc

_MARGIN = 6.0
_EPSILON = 2.0
_DIM = 128
_EMB_RANGE = (_MARGIN + _EPSILON) / _DIM
_BATCH = 16384
_ENT_D = 2 * _DIM
_NREL = 1000

_NC = 2
_NS = 16
_NW = _NC * _NS
_LANES = 16

_U = 64
_PER_W = _BATCH // _NW
_NU = _PER_W // _U

_SIN_C = (9.99999600e-01, -1.66665526e-01, 8.33240285e-03, -1.98086298e-04,
          2.69971060e-06, -2.03620814e-08)
_COS_C = (9.99999989e-01, -4.99999891e-01, 4.16664892e-02, -1.38878034e-03,
          2.47698803e-05, -2.70789985e-07, 1.72449738e-09)


def _poly_sin(x, t):
    acc = jnp.float32(_SIN_C[-1])
    for c in _SIN_C[-2::-1]:
        acc = acc * t + jnp.float32(c)
    return x * acc


def _poly_cos(t):
    acc = jnp.float32(_COS_C[-1])
    for c in _COS_C[-2::-1]:
        acc = acc * t + jnp.float32(c)
    return acc


def _trig_table_kernel(rel_ref, o_ref):
    phase = rel_ref[...] * np.float32(np.pi / _EMB_RANGE)
    t2 = phase * phase
    o_ref[:, :_DIM] = _poly_cos(t2)
    o_ref[:, _DIM:] = _poly_sin(phase, t2)


def _trig_table(relation_embedding):
    return pl.pallas_call(
        _trig_table_kernel,
        out_shape=jax.ShapeDtypeStruct((_NREL, _ENT_D), jnp.float32),
    )(relation_embedding)


def _sc_score_kernel(ent_hbm, trig_hbm, ih_hbm, it_hbm, ir_hbm, out_part,
                     idx_v, hb, tb, gb, part, gsem, wsem):
    wid = lax.axis_index("s") * _NC + lax.axis_index("c")
    base = wid * _PER_W
    pltpu.sync_copy(ih_hbm.at[pl.ds(base, _PER_W)],
                    idx_v.at[pl.ds(0, _PER_W)])
    pltpu.sync_copy(it_hbm.at[pl.ds(base, _PER_W)],
                    idx_v.at[pl.ds(_PER_W, _PER_W)])
    pltpu.sync_copy(ir_hbm.at[pl.ds(base, _PER_W)],
                    idx_v.at[pl.ds(2 * _PER_W, _PER_W)])

    def start_unit(u):
        b = u % 2
        off = u * _U
        return (
            pltpu.async_copy(ent_hbm.at[idx_v.at[pl.ds(off, _U)]],
                             hb[b], gsem),
            pltpu.async_copy(ent_hbm.at[idx_v.at[pl.ds(_PER_W + off, _U)]],
                             tb[b], gsem),
            pltpu.async_copy(trig_hbm.at[idx_v.at[pl.ds(2 * _PER_W + off,
                                                        _U)]],
                             gb[b], gsem),
        )

    gh = [None] * _NU
    wh = [None] * _NU
    gh[0] = start_unit(0)
    for u in range(_NU):
        if u + 1 < _NU:
            gh[u + 1] = start_unit(u + 1)
        for h in gh[u]:
            h.wait()
        if u >= 2:
            wh[u - 2].wait()
        b = u % 2
        hbuf, tbuf, gbuf, pbuf = hb[b], tb[b], gb[b], part[b]

        @pl.loop(0, _U)
        def _(r):
            acc = jnp.zeros((_LANES,), jnp.float32)
            for c in range(_DIM // _LANES):
                lo = c * _LANES
                hi = _DIM + lo
                re_h = hbuf[r, pl.ds(lo, _LANES)]
                im_h = hbuf[r, pl.ds(hi, _LANES)]
                re_t = tbuf[r, pl.ds(lo, _LANES)]
                im_t = tbuf[r, pl.ds(hi, _LANES)]
                cr = gbuf[r, pl.ds(lo, _LANES)]
                sr = gbuf[r, pl.ds(hi, _LANES)]
                d_re = re_h * cr - im_h * sr - re_t
                d_im = re_h * sr + im_h * cr - im_t
                s = d_re * d_re + d_im * d_im
                acc = acc + s * s
            pbuf[r, :] = acc

        wh[u] = pltpu.async_copy(
            pbuf, out_part.at[pl.ds(base + u * _U, _U)], wsem)
    if _NU >= 2:
        wh[_NU - 2].wait()
    wh[_NU - 1].wait()


def _sc_score(entity_embedding, trig, heads, tails, relations):
    mesh = plsc.VectorSubcoreMesh(core_axis_name="c", subcore_axis_name="s")
    run = pl.kernel(
        _sc_score_kernel,
        out_type=jax.ShapeDtypeStruct((_BATCH, _LANES), jnp.float32),
        mesh=mesh,
        scratch_types=[
            pltpu.VMEM((3 * _PER_W,), jnp.int32),
            tuple(pltpu.VMEM((_U, _ENT_D), jnp.float32) for _ in range(2)),
            tuple(pltpu.VMEM((_U, _ENT_D), jnp.float32) for _ in range(2)),
            tuple(pltpu.VMEM((_U, _ENT_D), jnp.float32) for _ in range(2)),
            tuple(pltpu.VMEM((_U, _LANES), jnp.float32) for _ in range(2)),
            pltpu.SemaphoreType.DMA,
            pltpu.SemaphoreType.DMA,
        ],
    )
    return run(entity_embedding, trig, heads, tails, relations)


_BB = 4096


def _finish_kernel(p_ref, o_ref):
    o_ref[...] = _MARGIN - jnp.sqrt(jnp.sum(p_ref[...], axis=1))


def _finish(part):
    nblk = _BATCH // _BB
    return pl.pallas_call(
        _finish_kernel,
        grid=(nblk,),
        in_specs=[pl.BlockSpec((_BB, _LANES), lambda i: (i, 0))],
        out_specs=pl.BlockSpec((_BB,), lambda i: (i,)),
        out_shape=jax.ShapeDtypeStruct((_BATCH,), jnp.float32),
        compiler_params=pltpu.CompilerParams(
            dimension_semantics=("parallel",)),
    )(part)


@jax.jit
def kernel(heads, relations, tails, entity_embedding, relation_embedding):
    heads = heads.astype(jnp.int32)
    tails = tails.astype(jnp.int32)
    relations = relations.astype(jnp.int32)
    trig = _trig_table(relation_embedding)
    part = _sc_score(entity_embedding, trig, heads, tails, relations)
    return _finish(part)


# speedup vs baseline: 1.3008x; 1.0006x over previous
import functools

import jax
import jax.numpy as jnp
import numpy as np
from jax import lax
from jax.experimental import pallas as pl
from jax.experimental.pallas import tpu as pltpu
from jax.experimental.pallas import tpu_sc as plsc

_MARGIN = 6.0
_EPSILON = 2.0
_DIM = 128
_EMB_RANGE = (_MARGIN + _EPSILON) / _DIM
_BATCH = 16384
_ENT_D = 2 * _DIM
_NREL = 1000

_NC = 2
_NS = 16
_NW = _NC * _NS
_LANES = 16

_U = 64
_PER_W = _BATCH // _NW
_NU = _PER_W // _U

_SIN_C = (9.99999600e-01, -1.66665526e-01, 8.33240285e-03, -1.98086298e-04,
          2.69971060e-06, -2.03620814e-08)
_COS_C = (9.99999989e-01, -4.99999891e-01, 4.16664892e-02, -1.38878034e-03,
          2.47698803e-05, -2.70789985e-07, 1.72449738e-09)


def _poly_sin(x, t):
    acc = jnp.float32(_SIN_C[-1])
    for c in _SIN_C[-2::-1]:
        acc = acc * t + jnp.float32(c)
    return x * acc


def _poly_cos(t):
    acc = jnp.float32(_COS_C[-1])
    for c in _COS_C[-2::-1]:
        acc = acc * t + jnp.float32(c)
    return acc


def _trig_table_kernel(rel_ref, o_ref):
    phase = rel_ref[...] * np.float32(np.pi / _EMB_RANGE)
    t2 = phase * phase
    o_ref[:, :_DIM] = _poly_cos(t2)
    o_ref[:, _DIM:] = _poly_sin(phase, t2)


def _trig_table(relation_embedding):
    return pl.pallas_call(
        _trig_table_kernel,
        out_shape=jax.ShapeDtypeStruct((_NREL, _ENT_D), jnp.float32),
    )(relation_embedding)


def _sc_score_kernel(ent_hbm, trig_hbm, ih_hbm, it_hbm, ir_hbm, out_part,
                     idx_v, hb, tb, gb, part, gsem, wsem):
    wid = lax.axis_index("s") * _NC + lax.axis_index("c")
    base = wid * _PER_W
    pltpu.sync_copy(ih_hbm.at[pl.ds(base, _PER_W)],
                    idx_v.at[pl.ds(0, _PER_W)])
    pltpu.sync_copy(it_hbm.at[pl.ds(base, _PER_W)],
                    idx_v.at[pl.ds(_PER_W, _PER_W)])
    pltpu.sync_copy(ir_hbm.at[pl.ds(base, _PER_W)],
                    idx_v.at[pl.ds(2 * _PER_W, _PER_W)])

    def start_unit(u):
        b = u % 2
        off = u * _U
        return (
            pltpu.async_copy(ent_hbm.at[idx_v.at[pl.ds(off, _U)]],
                             hb[b], gsem),
            pltpu.async_copy(ent_hbm.at[idx_v.at[pl.ds(_PER_W + off, _U)]],
                             tb[b], gsem),
            pltpu.async_copy(trig_hbm.at[idx_v.at[pl.ds(2 * _PER_W + off,
                                                        _U)]],
                             gb[b], gsem),
        )

    gh = [None] * _NU
    wh = [None] * _NU
    gh[0] = start_unit(0)
    for u in range(_NU):
        if u + 1 < _NU:
            gh[u + 1] = start_unit(u + 1)
        for h in gh[u]:
            h.wait()
        if u >= 2:
            wh[u - 2].wait()
        b = u % 2
        hbuf, tbuf, gbuf, pbuf = hb[b], tb[b], gb[b], part[b]

        @pl.loop(0, _U)
        def _(r):
            acc = jnp.zeros((_LANES,), jnp.float32)
            for c in range(_DIM // _LANES):
                lo = c * _LANES
                hi = _DIM + lo
                re_h = hbuf[r, pl.ds(lo, _LANES)]
                im_h = hbuf[r, pl.ds(hi, _LANES)]
                re_t = tbuf[r, pl.ds(lo, _LANES)]
                im_t = tbuf[r, pl.ds(hi, _LANES)]
                cr = gbuf[r, pl.ds(lo, _LANES)]
                sr = gbuf[r, pl.ds(hi, _LANES)]
                d_re = re_h * cr - im_h * sr - re_t
                d_im = re_h * sr + im_h * cr - im_t
                s = d_re * d_re + d_im * d_im
                acc = acc + s * s
            pbuf[r, :] = acc

        wh[u] = pltpu.async_copy(
            pbuf, out_part.at[pl.ds(base + u * _U, _U)], wsem)
    if _NU >= 2:
        wh[_NU - 2].wait()
    wh[_NU - 1].wait()


def _sc_score(entity_embedding, trig, heads, tails, relations):
    mesh = plsc.VectorSubcoreMesh(core_axis_name="c", subcore_axis_name="s")
    run = pl.kernel(
        _sc_score_kernel,
        out_type=jax.ShapeDtypeStruct((_BATCH, _LANES), jnp.float32),
        mesh=mesh,
        scratch_types=[
            pltpu.VMEM((3 * _PER_W,), jnp.int32),
            tuple(pltpu.VMEM((_U, _ENT_D), jnp.float32) for _ in range(2)),
            tuple(pltpu.VMEM((_U, _ENT_D), jnp.float32) for _ in range(2)),
            tuple(pltpu.VMEM((_U, _ENT_D), jnp.float32) for _ in range(2)),
            tuple(pltpu.VMEM((_U, _LANES), jnp.float32) for _ in range(2)),
            pltpu.SemaphoreType.DMA,
            pltpu.SemaphoreType.DMA,
        ],
    )
    return run(entity_embedding, trig, heads, tails, relations)


_GRP = 128 // _LANES


def _finish_kernel(p_ref, o_ref):
    x = p_ref[...]
    k = lax.broadcasted_iota(jnp.int32, (128, _GRP), 0) // _LANES
    g = lax.broadcasted_iota(jnp.int32, (128, _GRP), 1)
    m = (k == g).astype(jnp.bfloat16)
    s = jax.lax.dot_general(x.astype(jnp.bfloat16), m, (((1,), (0,)), ((), ())),
                            preferred_element_type=jnp.float32)
    o_ref[...] = _MARGIN - jnp.sqrt(s)


def _finish(part):
    p2 = part.reshape(_BATCH // _GRP, 128)
    out = pl.pallas_call(
        _finish_kernel,
        out_shape=jax.ShapeDtypeStruct((_BATCH // _GRP, _GRP), jnp.float32),
    )(p2)
    return out.reshape(_BATCH)


@jax.jit
def kernel(heads, relations, tails, entity_embedding, relation_embedding):
    heads = heads.astype(jnp.int32)
    tails = tails.astype(jnp.int32)
    relations = relations.astype(jnp.int32)
    trig = _trig_table(relation_embedding)
    part = _sc_score(entity_embedding, trig, heads, tails, relations)
    return _finish(part)


# speedup vs baseline: 1.4820x; 1.1393x over previous
import functools

import jax
import jax.numpy as jnp
import numpy as np
from jax import lax
from jax.experimental import pallas as pl
from jax.experimental.pallas import tpu as pltpu
from jax.experimental.pallas import tpu_sc as plsc

_MARGIN = 6.0
_EPSILON = 2.0
_DIM = 128
_EMB_RANGE = (_MARGIN + _EPSILON) / _DIM
_BATCH = 16384
_ENT_D = 2 * _DIM
_NREL = 1000

_NC = 2
_NS = 16
_NW = _NC * _NS
_LANES = 16

_U = 64
_PER_W = _BATCH // _NW
_NU = _PER_W // _U

_SIN_C = (9.99999600e-01, -1.66665526e-01, 8.33240285e-03, -1.98086298e-04,
          2.69971060e-06, -2.03620814e-08)
_COS_C = (9.99999989e-01, -4.99999891e-01, 4.16664892e-02, -1.38878034e-03,
          2.47698803e-05, -2.70789985e-07, 1.72449738e-09)


def _poly_sin(x, t):
    acc = jnp.float32(_SIN_C[-1])
    for c in _SIN_C[-2::-1]:
        acc = acc * t + jnp.float32(c)
    return x * acc


def _poly_cos(t):
    acc = jnp.float32(_COS_C[-1])
    for c in _COS_C[-2::-1]:
        acc = acc * t + jnp.float32(c)
    return acc


def _trig_table_kernel(rel_ref, o_ref):
    phase = rel_ref[...] * np.float32(np.pi / _EMB_RANGE)
    t2 = phase * phase
    o_ref[:, :_DIM] = _poly_cos(t2)
    o_ref[:, _DIM:] = _poly_sin(phase, t2)


def _trig_table(relation_embedding):
    return pl.pallas_call(
        _trig_table_kernel,
        out_shape=jax.ShapeDtypeStruct((_NREL, _ENT_D), jnp.float32),
    )(relation_embedding)


def _sc_score_kernel(ent_hbm, trig_hbm, ih_hbm, it_hbm, ir_hbm, out_part,
                     idx_v, hb, tb, gb, part, gsem, wsem):
    wid = lax.axis_index("s") * _NC + lax.axis_index("c")
    base = wid * _PER_W
    pltpu.sync_copy(ih_hbm.at[pl.ds(base, _PER_W)],
                    idx_v.at[pl.ds(0, _PER_W)])
    pltpu.sync_copy(it_hbm.at[pl.ds(base, _PER_W)],
                    idx_v.at[pl.ds(_PER_W, _PER_W)])
    pltpu.sync_copy(ir_hbm.at[pl.ds(base, _PER_W)],
                    idx_v.at[pl.ds(2 * _PER_W, _PER_W)])

    def start_unit(u):
        b = u % 2
        off = u * _U
        return (
            pltpu.async_copy(ent_hbm.at[idx_v.at[pl.ds(off, _U)]],
                             hb[b], gsem),
            pltpu.async_copy(ent_hbm.at[idx_v.at[pl.ds(_PER_W + off, _U)]],
                             tb[b], gsem),
            pltpu.async_copy(trig_hbm.at[idx_v.at[pl.ds(2 * _PER_W + off,
                                                        _U)]],
                             gb[b], gsem),
        )

    gh = [None] * _NU
    wh = [None] * _NU
    gh[0] = start_unit(0)
    for u in range(_NU):
        if u + 1 < _NU:
            gh[u + 1] = start_unit(u + 1)
        for h in gh[u]:
            h.wait()
        if u >= 2:
            wh[u - 2].wait()
        b = u % 2
        hbuf, tbuf, gbuf, pbuf = hb[b], tb[b], gb[b], part[b]

        @pl.loop(0, _U)
        def _(r):
            acc = jnp.zeros((_LANES,), jnp.float32)
            for c in range(_DIM // _LANES):
                lo = c * _LANES
                hi = _DIM + lo
                re_h = hbuf[r, pl.ds(lo, _LANES)]
                im_h = hbuf[r, pl.ds(hi, _LANES)]
                re_t = tbuf[r, pl.ds(lo, _LANES)]
                im_t = tbuf[r, pl.ds(hi, _LANES)]
                cr = gbuf[r, pl.ds(lo, _LANES)]
                sr = gbuf[r, pl.ds(hi, _LANES)]
                d_re = re_h * cr - im_h * sr - re_t
                d_im = re_h * sr + im_h * cr - im_t
                s = d_re * d_re + d_im * d_im
                acc = acc + s * s
            pbuf[pl.ds(r * _LANES, _LANES)] = acc

        wh[u] = pltpu.async_copy(
            pbuf, out_part.at[pl.ds((base + u * _U) * _LANES, _U * _LANES)],
            wsem)
    if _NU >= 2:
        wh[_NU - 2].wait()
    wh[_NU - 1].wait()


def _sc_score(entity_embedding, trig, heads, tails, relations):
    mesh = plsc.VectorSubcoreMesh(core_axis_name="c", subcore_axis_name="s")
    run = pl.kernel(
        _sc_score_kernel,
        out_type=jax.ShapeDtypeStruct((_BATCH * _LANES,), jnp.float32),
        mesh=mesh,
        scratch_types=[
            pltpu.VMEM((3 * _PER_W,), jnp.int32),
            tuple(pltpu.VMEM((_U, _ENT_D), jnp.float32) for _ in range(2)),
            tuple(pltpu.VMEM((_U, _ENT_D), jnp.float32) for _ in range(2)),
            tuple(pltpu.VMEM((_U, _ENT_D), jnp.float32) for _ in range(2)),
            tuple(pltpu.VMEM((_U * _LANES,), jnp.float32) for _ in range(2)),
            pltpu.SemaphoreType.DMA,
            pltpu.SemaphoreType.DMA,
        ],
    )
    return run(entity_embedding, trig, heads, tails, relations)


_GRP = 128 // _LANES


def _finish_kernel(p_ref, o_ref):
    x = p_ref[...]
    k = lax.broadcasted_iota(jnp.int32, (128, _GRP), 0) // _LANES
    g = lax.broadcasted_iota(jnp.int32, (128, _GRP), 1)
    m = (k == g).astype(jnp.bfloat16)
    s = jax.lax.dot_general(x.astype(jnp.bfloat16), m, (((1,), (0,)), ((), ())),
                            preferred_element_type=jnp.float32)
    o_ref[...] = _MARGIN - jnp.sqrt(s)


def _finish(part):
    p2 = part.reshape(_BATCH // _GRP, 128)
    out = pl.pallas_call(
        _finish_kernel,
        out_shape=jax.ShapeDtypeStruct((_BATCH // _GRP, _GRP), jnp.float32),
    )(p2)
    return out.reshape(_BATCH)


@jax.jit
def kernel(heads, relations, tails, entity_embedding, relation_embedding):
    heads = heads.astype(jnp.int32)
    tails = tails.astype(jnp.int32)
    relations = relations.astype(jnp.int32)
    trig = _trig_table(relation_embedding)
    part = _sc_score(entity_embedding, trig, heads, tails, relations)
    return _finish(part)
